# Initial kernel scaffold; baseline (speedup 1.0000x reference)
import jax
import jax.numpy as jnp
from jax.experimental import pallas as pl


def kernel(x, w_t, b2d):
    raise NotImplementedError("write your pallas kernel here")


# speedup vs baseline: 4.4350x; 4.4350x over previous
import functools

import jax
import jax.numpy as jnp
from jax.experimental import pallas as pl
from jax.experimental.pallas import tpu as pltpu


def _linear_sigmoid_kernel(x_ref, w_ref, b_ref, o_ref):
    acc = jnp.dot(
        x_ref[...].astype(jnp.bfloat16),
        w_ref[...],
        preferred_element_type=jnp.float32,
    )
    o_ref[...] = jax.nn.sigmoid(acc + b_ref[...])


@jax.jit
def kernel(x, w_t, b2d):
    x = x.astype(jnp.float32)
    M, K = x.shape
    K2, N = w_t.shape
    assert K == K2 and b2d.shape == (1, N)

    w_bf = w_t.astype(jnp.bfloat16)
    b2d = b2d.astype(jnp.float32)

    tm = 512
    while M % tm != 0 and tm > 8:
        tm //= 2
    m_pad = M
    if M % tm != 0:
        m_pad = ((M + tm - 1) // tm) * tm
        x = jnp.pad(x, ((0, m_pad - M), (0, 0)))

    out = pl.pallas_call(
        _linear_sigmoid_kernel,
        out_shape=jax.ShapeDtypeStruct((m_pad, N), jnp.float32),
        grid=(m_pad // tm,),
        in_specs=[
            pl.BlockSpec((tm, K), lambda i: (i, 0)),
            pl.BlockSpec((K, N), lambda i: (0, 0)),
            pl.BlockSpec((1, N), lambda i: (0, 0)),
        ],
        out_specs=pl.BlockSpec((tm, N), lambda i: (i, 0)),
        compiler_params=pltpu.CompilerParams(
            dimension_semantics=("parallel",),
        ),
    )(x, w_bf, b2d)

    if m_pad != M:
        out = out[:M]
    return out


# speedup vs baseline: 4.9990x; 1.1272x over previous
import functools

import jax
import jax.numpy as jnp
from jax.experimental import pallas as pl
from jax.experimental.pallas import tpu as pltpu


def _linear_sigmoid_kernel(x_ref, w_ref, b_ref, o_ref):
    acc = jnp.dot(
        x_ref[...].astype(jnp.bfloat16),
        w_ref[...],
        preferred_element_type=jnp.float32,
    )
    o_ref[...] = jax.nn.sigmoid(acc + b_ref[...])


@jax.jit
def kernel(x, w_t, b2d):
    x = x.astype(jnp.float32)
    M, K = x.shape
    K2, N = w_t.shape
    assert K == K2 and b2d.shape == (1, N)

    w_bf = w_t.astype(jnp.bfloat16)
    b2d = b2d.astype(jnp.float32)

    tm = 1024
    while M % tm != 0 and tm > 8:
        tm //= 2
    m_pad = M
    if M % tm != 0:
        m_pad = ((M + tm - 1) // tm) * tm
        x = jnp.pad(x, ((0, m_pad - M), (0, 0)))

    out = pl.pallas_call(
        _linear_sigmoid_kernel,
        out_shape=jax.ShapeDtypeStruct((m_pad, N), jnp.float32),
        grid=(m_pad // tm,),
        in_specs=[
            pl.BlockSpec((tm, K), lambda i: (i, 0)),
            pl.BlockSpec((K, N), lambda i: (0, 0)),
            pl.BlockSpec((1, N), lambda i: (0, 0)),
        ],
        out_specs=pl.BlockSpec((tm, N), lambda i: (i, 0)),
        compiler_params=pltpu.CompilerParams(
            dimension_semantics=("parallel",),
        ),
    )(x, w_bf, b2d)

    if m_pad != M:
        out = out[:M]
    return out
